# baseline (speedup 1.0000x reference)
import functools

import jax
import jax.numpy as jnp
from jax import lax
from jax.experimental import pallas as pl
from jax.experimental.pallas import tpu as pltpu
from jax.experimental.pallas import tpu_sc as plsc

N = 10000
E = 320000
D = 128
NPAD = 10240
ROWS_PER_TILE = NPAD // 16
CH = 96
EDGES_PER_TILE = E // 32
NCHUNK = -(-EDGES_PER_TILE // CH)
PAD_PER_TILE = NCHUNK * CH - EDGES_PER_TILE
REAL_GROUPS_LAST = (EDGES_PER_TILE - (NCHUNK - 1) * CH) // 16


def _head1_body(x_ref, w_ref, as_ref, ad_ref, h_ref, sas_ref, sad_ref):
    h = jnp.dot(x_ref[...], w_ref[...], preferred_element_type=jnp.float32)
    h_ref[...] = h
    sas_ref[...] = jnp.sum(h * as_ref[...], axis=1)
    sad_ref[...] = jnp.sum(h * ad_ref[...], axis=1)


def _mid_body(a0_ref, a1_ref, d0_ref, d1_ref, b_ref, w_ref, as_ref, ad_ref,
              h_ref, sas_ref, sad_ref):
    den = d0_ref[:N, :] + d1_ref[:N, :]
    rec = 1.0 / (den + 1e-16)
    s = a0_ref[:N, :] + a1_ref[:N, :]
    z = s * rec + b_ref[...]
    z = jnp.where(z > 0.0, z, jnp.exp(z) - 1.0)
    h = jnp.dot(z, w_ref[...], preferred_element_type=jnp.float32)
    h_ref[...] = h
    sas_ref[...] = jnp.sum(h * as_ref[...], axis=1)
    sad_ref[...] = jnp.sum(h * ad_ref[...], axis=1)


def _final_body(a0_ref, a1_ref, d0_ref, d1_ref, b_ref, o_ref):
    den = d0_ref[:N, :] + d1_ref[:N, :]
    rec = 1.0 / (den + 1e-16)
    s = a0_ref[:N, :] + a1_ref[:N, :]
    o_ref[...] = s * rec + b_ref[...]


_OUT_HEAD = [
    jax.ShapeDtypeStruct((N, D), jnp.float32),
    jax.ShapeDtypeStruct((N,), jnp.float32),
    jax.ShapeDtypeStruct((N,), jnp.float32),
]

_head1 = pl.pallas_call(_head1_body, out_shape=_OUT_HEAD)
_mid = pl.pallas_call(_mid_body, out_shape=_OUT_HEAD)
_final = pl.pallas_call(
    _final_body, out_shape=jax.ShapeDtypeStruct((N, D), jnp.float32)
)


def _vmax_all(ref):
    def body(i, m):
        return jnp.maximum(m, ref[pl.ds(i * 16, 16)])
    m = lax.fori_loop(0, N // 16, body, jnp.full((16,), -jnp.inf, jnp.float32))
    s = m[0]
    for i in range(1, 16):
        s = jnp.maximum(s, m[i])
    return s


@functools.lru_cache(maxsize=None)
def _make_sc_edge():
    mesh = plsc.VectorSubcoreMesh(
        core_axis_name="c", subcore_axis_name="s", num_cores=2, num_subcores=16
    )

    @functools.partial(
        pl.kernel,
        out_type=(
            jax.ShapeDtypeStruct((NPAD, D), jnp.float32),
            jax.ShapeDtypeStruct((NPAD, D), jnp.float32),
            jax.ShapeDtypeStruct((NPAD,), jnp.float32),
            jax.ShapeDtypeStruct((NPAD,), jnp.float32),
        ),
        mesh=mesh,
        compiler_params=pltpu.CompilerParams(needs_layout_passes=False),
        scratch_types=dict(
            asb=pltpu.VMEM((N,), jnp.float32),
            adb=pltpu.VMEM((N,), jnp.float32),
            srcA=pltpu.VMEM((CH,), jnp.int32),
            srcB=pltpu.VMEM((CH,), jnp.int32),
            dstA=pltpu.VMEM((CH,), jnp.int32),
            dstB=pltpu.VMEM((CH,), jnp.int32),
            sidxA=pltpu.VMEM((CH,), jnp.int32),
            sidxB=pltpu.VMEM((CH,), jnp.int32),
            exA=pltpu.VMEM((CH,), jnp.float32),
            exB=pltpu.VMEM((CH,), jnp.float32),
            rowA=pltpu.VMEM((CH, D), jnp.float32),
            rowB=pltpu.VMEM((CH, D), jnp.float32),
            acc_sh=pltpu.VMEM_SHARED((NPAD, D), jnp.float32),
            den_sh=pltpu.VMEM_SHARED((NPAD,), jnp.float32),
            psem=pltpu.SemaphoreType.DMA,
            gsem=pltpu.SemaphoreType.DMA,
            rsem=pltpu.SemaphoreType.DMA,
            dsem=pltpu.SemaphoreType.DMA,
        ),
    )
    def sc_edge(h, asv, adv, src3, dst3, acc0, acc1, den0, den1,
                asb, adb, srcA, srcB, dstA, dstB, sidxA, sidxB,
                exA, exB, rowA, rowB,
                acc_sh, den_sh, psem, gsem, rsem, dsem):
        cidx = lax.axis_index("c")
        sidx = lax.axis_index("s")
        wid = cidx * 16 + sidx

        pltpu.sync_copy(asv, asb)
        pltpu.sync_copy(adv, adb)

        zeros16 = jnp.zeros((16,), jnp.float32)

        def zrow(j, c):
            for k in range(D // 16):
                rowA[j, pl.ds(k * 16, 16)] = zeros16
            return c
        lax.fori_loop(0, CH, zrow, 0)

        def zex(j, c):
            exA[pl.ds(j * 16, 16)] = zeros16
            return c
        lax.fori_loop(0, CH // 16, zex, 0)

        r0 = sidx * ROWS_PER_TILE
        for jb in range(ROWS_PER_TILE // 80):
            rb = pl.multiple_of(r0 + jb * 80, 8)
            pltpu.sync_copy(rowA.at[pl.ds(0, 80)], acc_sh.at[pl.ds(rb, 80)])
            pltpu.sync_copy(exA.at[pl.ds(0, 80)], den_sh.at[pl.ds(rb, 80)])

        m_as = _vmax_all(asb)
        m_ad = _vmax_all(adb)
        msum = m_as + m_ad
        mbound = jnp.where(msum >= 0.0, msum, 0.2 * msum)

        plsc.subcore_barrier()

        esrc = src3.at[wid]
        edst = dst3.at[wid]

        def do_chunk(k, su, du, xu, iu, ru, sv, dv, xv, iv, rv, tail):
            def exgrp(j, c2):
                svv = su[pl.ds(j * 16, 16)]
                dvv = du[pl.ds(j * 16, 16)]
                av = plsc.load_gather(asb, [svv])
                bv = plsc.load_gather(adb, [dvv])
                e = av + bv
                e = jnp.where(e >= 0.0, e, 0.2 * e) - mbound
                xu[pl.ds(j * 16, 16)] = jnp.exp(e)
                return c2
            lax.fori_loop(0, CH // 16, exgrp, 0)

            @pl.when(k == NCHUNK - 1)
            def _():
                for j in range(REAL_GROUPS_LAST, CH // 16):
                    xu[pl.ds(j * 16, 16)] = zeros16

            for j in range(CH // 16):
                iu[pl.ds(j * 16, 16)] = du[pl.ds(j * 16, 16)]

            @pl.when(k >= 1)
            def _():
                pltpu.make_async_copy(rv, acc_sh.at[iv], rsem).wait()
                pltpu.make_async_copy(xv, den_sh.at[iv], dsem).wait()

            if not tail:
                pltpu.make_async_copy(esrc.at[k + 1], sv, psem).wait()
                pltpu.make_async_copy(edst.at[k + 1], dv, psem).wait()
                pltpu.async_copy(h.at[sv], rv, gsem)

            pltpu.make_async_copy(h.at[su], ru, gsem).wait()

            if not tail:
                @pl.when(k + 2 < NCHUNK)
                def _():
                    pltpu.async_copy(esrc.at[k + 2], su, psem)
                    pltpu.async_copy(edst.at[k + 2], du, psem)

            def sblk(q, c2):
                ex16 = xu[pl.ds(q * 16, 16)]
                for jj in range(16):
                    j = q * 16 + jj
                    s_ = ex16[jj]
                    for kk in range(D // 16):
                        ru[j, pl.ds(kk * 16, 16)] = (
                            ru[j, pl.ds(kk * 16, 16)] * s_
                        )
                return c2
            lax.fori_loop(0, CH // 16, sblk, 0)

            pltpu.async_copy(xu, den_sh.at[iu], dsem, add=True)
            pltpu.async_copy(ru, acc_sh.at[iu], rsem, add=True)

        pltpu.async_copy(esrc.at[0], srcA, psem)
        pltpu.async_copy(edst.at[0], dstA, psem)
        pltpu.async_copy(esrc.at[1], srcB, psem)
        pltpu.async_copy(edst.at[1], dstB, psem)
        pltpu.make_async_copy(esrc.at[0], srcA, psem).wait()
        pltpu.make_async_copy(edst.at[0], dstA, psem).wait()
        pltpu.async_copy(h.at[srcA], rowA, gsem)

        def pair(p, c):
            k = p * 2
            do_chunk(k, srcA, dstA, exA, sidxA, rowA,
                     srcB, dstB, exB, sidxB, rowB, False)
            do_chunk(k + 1, srcB, dstB, exB, sidxB, rowB,
                     srcA, dstA, exA, sidxA, rowA, False)
            return c
        lax.fori_loop(0, NCHUNK // 2, pair, 0)

        do_chunk(NCHUNK - 1, srcA, dstA, exA, sidxA, rowA,
                 srcB, dstB, exB, sidxB, rowB, True)
        pltpu.make_async_copy(rowA, acc_sh.at[sidxA], rsem).wait()
        pltpu.make_async_copy(exA, den_sh.at[sidxA], dsem).wait()

        plsc.subcore_barrier()

        rr = pl.multiple_of(r0, 8)

        @pl.when(cidx == 0)
        def _():
            pltpu.sync_copy(acc_sh.at[pl.ds(rr, ROWS_PER_TILE)],
                            acc0.at[pl.ds(rr, ROWS_PER_TILE)])
            pltpu.sync_copy(den_sh.at[pl.ds(rr, ROWS_PER_TILE)],
                            den0.at[pl.ds(rr, ROWS_PER_TILE)])

        @pl.when(cidx == 1)
        def _():
            pltpu.sync_copy(acc_sh.at[pl.ds(rr, ROWS_PER_TILE)],
                            acc1.at[pl.ds(rr, ROWS_PER_TILE)])
            pltpu.sync_copy(den_sh.at[pl.ds(rr, ROWS_PER_TILE)],
                            den1.at[pl.ds(rr, ROWS_PER_TILE)])

    return sc_edge


def kernel(x, edge_index, W0, a_src0, a_dst0, b0, W1, a_src1, a_dst1, b1):
    pad = jnp.zeros((32, PAD_PER_TILE), jnp.int32)
    src3 = jnp.concatenate(
        [edge_index[0].astype(jnp.int32).reshape(32, EDGES_PER_TILE), pad],
        axis=1).reshape(32, NCHUNK, CH)
    dst3 = jnp.concatenate(
        [edge_index[1].astype(jnp.int32).reshape(32, EDGES_PER_TILE), pad],
        axis=1).reshape(32, NCHUNK, CH)
    sc_edge = _make_sc_edge()
    h0, as0, ad0 = _head1(x, W0, a_src0, a_dst0)
    a0, a1, d0, d1 = sc_edge(h0, as0, ad0, src3, dst3)
    h1, as1, ad1 = _mid(a0, a1, d0.reshape(NPAD, 1), d1.reshape(NPAD, 1),
                        b0, W1, a_src1, a_dst1)
    a0, a1, d0, d1 = sc_edge(h1, as1, ad1, src3, dst3)
    return _final(a0, a1, d0.reshape(NPAD, 1), d1.reshape(NPAD, 1), b1)


# speedup vs baseline: 1.4508x; 1.4508x over previous
import functools

import jax
import jax.numpy as jnp
from jax import lax
from jax.experimental import pallas as pl
from jax.experimental.pallas import tpu as pltpu
from jax.experimental.pallas import tpu_sc as plsc

N = 10000
E = 320000
D = 128
NPAD = 10240
ROWS_PER_TILE = NPAD // 16
CH = 80
EDGES_PER_TILE = E // 32
NCHUNK = -(-EDGES_PER_TILE // CH)
PAD_PER_TILE = NCHUNK * CH - EDGES_PER_TILE
REAL_GROUPS_LAST = (EDGES_PER_TILE - (NCHUNK - 1) * CH) // 16


def _head1_body(x_ref, w_ref, as_ref, ad_ref, h_ref, sas_ref, sad_ref):
    h = jnp.dot(x_ref[...], w_ref[...], preferred_element_type=jnp.float32)
    h_ref[...] = h
    sas_ref[...] = jnp.sum(h * as_ref[...], axis=1)
    sad_ref[...] = jnp.sum(h * ad_ref[...], axis=1)


def _mid_body(a0_ref, a1_ref, d0_ref, d1_ref, b_ref, w_ref, as_ref, ad_ref,
              h_ref, sas_ref, sad_ref):
    den = d0_ref[:N, :] + d1_ref[:N, :]
    rec = 1.0 / (den + 1e-16)
    s = a0_ref[:N, :] + a1_ref[:N, :]
    z = s * rec + b_ref[...]
    z = jnp.where(z > 0.0, z, jnp.exp(z) - 1.0)
    h = jnp.dot(z, w_ref[...], preferred_element_type=jnp.float32)
    h_ref[...] = h
    sas_ref[...] = jnp.sum(h * as_ref[...], axis=1)
    sad_ref[...] = jnp.sum(h * ad_ref[...], axis=1)


def _final_body(a0_ref, a1_ref, d0_ref, d1_ref, b_ref, o_ref):
    den = d0_ref[:N, :] + d1_ref[:N, :]
    rec = 1.0 / (den + 1e-16)
    s = a0_ref[:N, :] + a1_ref[:N, :]
    o_ref[...] = s * rec + b_ref[...]


_OUT_HEAD = [
    jax.ShapeDtypeStruct((N, D), jnp.float32),
    jax.ShapeDtypeStruct((N,), jnp.float32),
    jax.ShapeDtypeStruct((N,), jnp.float32),
]

_head1 = pl.pallas_call(_head1_body, out_shape=_OUT_HEAD)
_mid = pl.pallas_call(_mid_body, out_shape=_OUT_HEAD)
_final = pl.pallas_call(
    _final_body, out_shape=jax.ShapeDtypeStruct((N, D), jnp.float32)
)


def _vmax_all(ref):
    def body(i, m):
        return jnp.maximum(m, ref[pl.ds(i * 16, 16)])
    m = lax.fori_loop(0, N // 16, body, jnp.full((16,), -jnp.inf, jnp.float32))
    s = m[0]
    for i in range(1, 16):
        s = jnp.maximum(s, m[i])
    return s


@functools.lru_cache(maxsize=None)
def _make_sc_edge():
    mesh = plsc.VectorSubcoreMesh(
        core_axis_name="c", subcore_axis_name="s", num_cores=2, num_subcores=16
    )

    @functools.partial(
        pl.kernel,
        out_type=(
            jax.ShapeDtypeStruct((NPAD, D), jnp.float32),
            jax.ShapeDtypeStruct((NPAD, D), jnp.float32),
            jax.ShapeDtypeStruct((NPAD,), jnp.float32),
            jax.ShapeDtypeStruct((NPAD,), jnp.float32),
        ),
        mesh=mesh,
        compiler_params=pltpu.CompilerParams(needs_layout_passes=False),
        scratch_types=dict(
            asb=pltpu.VMEM((N,), jnp.float32),
            adb=pltpu.VMEM((N,), jnp.float32),
            srcA=pltpu.VMEM((CH,), jnp.int32),
            srcB=pltpu.VMEM((CH,), jnp.int32),
            dstA=pltpu.VMEM((CH,), jnp.int32),
            dstB=pltpu.VMEM((CH,), jnp.int32),
            sidxA=pltpu.VMEM((CH,), jnp.int32),
            sidxB=pltpu.VMEM((CH,), jnp.int32),
            exA=pltpu.VMEM((CH,), jnp.float32),
            exB=pltpu.VMEM((CH,), jnp.float32),
            rowA=pltpu.VMEM((CH, D), jnp.float32),
            rowB=pltpu.VMEM((CH, D), jnp.float32),
            acc_sh=pltpu.VMEM_SHARED((NPAD, D), jnp.float32),
            den_sh=pltpu.VMEM_SHARED((NPAD,), jnp.float32),
            psem=pltpu.SemaphoreType.DMA,
            gsem=pltpu.SemaphoreType.DMA,
            rsem=pltpu.SemaphoreType.DMA,
            dsem=pltpu.SemaphoreType.DMA,
        ),
    )
    def sc_edge(h, asv, adv, src3, dst3, acc0, acc1, den0, den1,
                asb, adb, srcA, srcB, dstA, dstB, sidxA, sidxB,
                exA, exB, rowA, rowB,
                acc_sh, den_sh, psem, gsem, rsem, dsem):
        cidx = lax.axis_index("c")
        sidx = lax.axis_index("s")
        wid = cidx * 16 + sidx

        pltpu.sync_copy(asv, asb)
        pltpu.sync_copy(adv, adb)

        zeros16 = jnp.zeros((16,), jnp.float32)

        def zrow(j, c):
            for k in range(D // 16):
                rowA[j, pl.ds(k * 16, 16)] = zeros16
            return c
        lax.fori_loop(0, CH, zrow, 0)

        def zex(j, c):
            exA[pl.ds(j * 16, 16)] = zeros16
            return c
        lax.fori_loop(0, CH // 16, zex, 0)

        r0 = sidx * ROWS_PER_TILE
        for jb in range(ROWS_PER_TILE // 80):
            rb = pl.multiple_of(r0 + jb * 80, 8)
            pltpu.sync_copy(rowA.at[pl.ds(0, 80)], acc_sh.at[pl.ds(rb, 80)])
            pltpu.sync_copy(exA.at[pl.ds(0, 80)], den_sh.at[pl.ds(rb, 80)])

        m_as = _vmax_all(asb)
        m_ad = _vmax_all(adb)
        msum = m_as + m_ad
        mbound = jnp.where(msum >= 0.0, msum, 0.2 * msum)

        plsc.subcore_barrier()

        esrc = src3.at[wid]
        edst = dst3.at[wid]

        def do_chunk(k, su, du, xu, iu, ru, sv, dv, xv, iv, rv, tail):
            def exgrp(j, c2):
                svv = su[pl.ds(j * 16, 16)]
                dvv = du[pl.ds(j * 16, 16)]
                av = plsc.load_gather(asb, [svv])
                bv = plsc.load_gather(adb, [dvv])
                e = av + bv
                e = jnp.where(e >= 0.0, e, 0.2 * e) - mbound
                xu[pl.ds(j * 16, 16)] = jnp.exp(e)
                return c2
            lax.fori_loop(0, CH // 16, exgrp, 0)

            @pl.when(k == NCHUNK - 1)
            def _():
                for j in range(REAL_GROUPS_LAST, CH // 16):
                    xu[pl.ds(j * 16, 16)] = zeros16

            for j in range(CH // 16):
                iu[pl.ds(j * 16, 16)] = du[pl.ds(j * 16, 16)]

            @pl.when(k >= 1)
            def _():
                pltpu.make_async_copy(rv, acc_sh.at[iv], rsem).wait()
                pltpu.make_async_copy(xv, den_sh.at[iv], dsem).wait()

            if not tail:
                pltpu.make_async_copy(esrc.at[k + 1], sv, psem).wait()
                pltpu.make_async_copy(edst.at[k + 1], dv, psem).wait()
                pltpu.async_copy(h.at[sv], rv, gsem)

            pltpu.make_async_copy(h.at[su], ru, gsem).wait()

            if not tail:
                @pl.when(k + 2 < NCHUNK)
                def _():
                    pltpu.async_copy(esrc.at[k + 2], su, psem)
                    pltpu.async_copy(edst.at[k + 2], du, psem)

            def sblk(q, c2):
                ex16 = xu[pl.ds(q * 16, 16)]
                for jj in range(16):
                    j = q * 16 + jj
                    s_ = ex16[jj]
                    for kk in range(D // 16):
                        ru[j, pl.ds(kk * 16, 16)] = (
                            ru[j, pl.ds(kk * 16, 16)] * s_
                        )
                return c2
            lax.fori_loop(0, CH // 16, sblk, 0)

            pltpu.async_copy(xu, den_sh.at[iu], dsem, add=True)
            pltpu.async_copy(ru, acc_sh.at[iu], rsem, add=True)

        pltpu.async_copy(esrc.at[0], srcA, psem)
        pltpu.async_copy(edst.at[0], dstA, psem)
        pltpu.async_copy(esrc.at[1], srcB, psem)
        pltpu.async_copy(edst.at[1], dstB, psem)
        pltpu.make_async_copy(esrc.at[0], srcA, psem).wait()
        pltpu.make_async_copy(edst.at[0], dstA, psem).wait()
        pltpu.async_copy(h.at[srcA], rowA, gsem)

        def pair(p, c):
            k = p * 2
            do_chunk(k, srcA, dstA, exA, sidxA, rowA,
                     srcB, dstB, exB, sidxB, rowB, False)
            do_chunk(k + 1, srcB, dstB, exB, sidxB, rowB,
                     srcA, dstA, exA, sidxA, rowA, False)
            return c
        lax.fori_loop(0, NCHUNK // 2, pair, 0)

        do_chunk(NCHUNK - 1, srcA, dstA, exA, sidxA, rowA,
                 srcB, dstB, exB, sidxB, rowB, True)
        pltpu.make_async_copy(rowA, acc_sh.at[sidxA], rsem).wait()
        pltpu.make_async_copy(exA, den_sh.at[sidxA], dsem).wait()

        plsc.subcore_barrier()

        rr = pl.multiple_of(r0, 8)

        @pl.when(cidx == 0)
        def _():
            pltpu.sync_copy(acc_sh.at[pl.ds(rr, ROWS_PER_TILE)],
                            acc0.at[pl.ds(rr, ROWS_PER_TILE)])
            pltpu.sync_copy(den_sh.at[pl.ds(rr, ROWS_PER_TILE)],
                            den0.at[pl.ds(rr, ROWS_PER_TILE)])

        @pl.when(cidx == 1)
        def _():
            pltpu.sync_copy(acc_sh.at[pl.ds(rr, ROWS_PER_TILE)],
                            acc1.at[pl.ds(rr, ROWS_PER_TILE)])
            pltpu.sync_copy(den_sh.at[pl.ds(rr, ROWS_PER_TILE)],
                            den1.at[pl.ds(rr, ROWS_PER_TILE)])

    return sc_edge


def kernel(x, edge_index, W0, a_src0, a_dst0, b0, W1, a_src1, a_dst1, b1):
    pad = jnp.zeros((32, PAD_PER_TILE), jnp.int32)
    src3 = jnp.concatenate(
        [edge_index[0].astype(jnp.int32).reshape(32, EDGES_PER_TILE), pad],
        axis=1).reshape(32, NCHUNK, CH)
    dst3 = jnp.concatenate(
        [edge_index[1].astype(jnp.int32).reshape(32, EDGES_PER_TILE), pad],
        axis=1).reshape(32, NCHUNK, CH)
    sc_edge = _make_sc_edge()
    h0, as0, ad0 = _head1(x, W0, a_src0, a_dst0)
    a0, a1, d0, d1 = sc_edge(h0, as0, ad0, src3, dst3)
    h1, as1, ad1 = _mid(a0, a1, d0.reshape(NPAD, 1), d1.reshape(NPAD, 1),
                        b0, W1, a_src1, a_dst1)
    a0, a1, d0, d1 = sc_edge(h1, as1, ad1, src3, dst3)
    return _final(a0, a1, d0.reshape(NPAD, 1), d1.reshape(NPAD, 1), b1)


# speedup vs baseline: 1.4932x; 1.0293x over previous
import functools

import jax
import jax.numpy as jnp
from jax import lax
from jax.experimental import pallas as pl
from jax.experimental.pallas import tpu as pltpu
from jax.experimental.pallas import tpu_sc as plsc

N = 10000
E = 320000
D = 128
NPAD = 10240
ROWS_PER_TILE = NPAD // 16
CH = 80
EDGES_PER_TILE = E // 32
NCHUNK = -(-EDGES_PER_TILE // CH)
PAD_PER_TILE = NCHUNK * CH - EDGES_PER_TILE
REAL_GROUPS_LAST = (EDGES_PER_TILE - (NCHUNK - 1) * CH) // 16


def _logits(h, a2):
    return lax.dot_general(a2, h, (((1,), (1,)), ((), ())),
                           preferred_element_type=jnp.float32)


def _head1_body(x_ref, w_ref, a2_ref, h_ref, sa_ref):
    h = jnp.dot(x_ref[...], w_ref[...], preferred_element_type=jnp.float32)
    h_ref[...] = h
    sa_ref[...] = _logits(h, a2_ref[...])


def _mid_body(a0_ref, a1_ref, d0_ref, d1_ref, b_ref, w_ref, a2_ref,
              h_ref, sa_ref):
    den = d0_ref[:N, :] + d1_ref[:N, :]
    rec = 1.0 / (den + 1e-16)
    s = a0_ref[:N, :] + a1_ref[:N, :]
    z = s * rec + b_ref[...]
    z = jnp.where(z > 0.0, z, jnp.exp(z) - 1.0)
    h = jnp.dot(z, w_ref[...], preferred_element_type=jnp.float32)
    h_ref[...] = h
    sa_ref[...] = _logits(h, a2_ref[...])


def _final_body(a0_ref, a1_ref, d0_ref, d1_ref, b_ref, o_ref):
    den = d0_ref[:N, :] + d1_ref[:N, :]
    rec = 1.0 / (den + 1e-16)
    s = a0_ref[:N, :] + a1_ref[:N, :]
    o_ref[...] = s * rec + b_ref[...]


_OUT_HEAD = [
    jax.ShapeDtypeStruct((N, D), jnp.float32),
    jax.ShapeDtypeStruct((2, N), jnp.float32),
]

_head1 = pl.pallas_call(_head1_body, out_shape=_OUT_HEAD)
_mid = pl.pallas_call(_mid_body, out_shape=_OUT_HEAD)
_final = pl.pallas_call(
    _final_body, out_shape=jax.ShapeDtypeStruct((N, D), jnp.float32)
)


def _vmax_all(ref):
    def body(i, m):
        return jnp.maximum(m, ref[pl.ds(i * 16, 16)])
    m = lax.fori_loop(0, N // 16, body, jnp.full((16,), -jnp.inf, jnp.float32))
    s = m[0]
    for i in range(1, 16):
        s = jnp.maximum(s, m[i])
    return s


@functools.lru_cache(maxsize=None)
def _make_sc_edge():
    mesh = plsc.VectorSubcoreMesh(
        core_axis_name="c", subcore_axis_name="s", num_cores=2, num_subcores=16
    )

    @functools.partial(
        pl.kernel,
        out_type=(
            jax.ShapeDtypeStruct((NPAD, D), jnp.float32),
            jax.ShapeDtypeStruct((NPAD, D), jnp.float32),
            jax.ShapeDtypeStruct((NPAD,), jnp.float32),
            jax.ShapeDtypeStruct((NPAD,), jnp.float32),
        ),
        mesh=mesh,
        compiler_params=pltpu.CompilerParams(needs_layout_passes=False),
        scratch_types=dict(
            asb=pltpu.VMEM((N,), jnp.float32),
            adb=pltpu.VMEM((N,), jnp.float32),
            srcA=pltpu.VMEM((CH,), jnp.int32),
            srcB=pltpu.VMEM((CH,), jnp.int32),
            dstA=pltpu.VMEM((CH,), jnp.int32),
            dstB=pltpu.VMEM((CH,), jnp.int32),
            sidxA=pltpu.VMEM((CH,), jnp.int32),
            sidxB=pltpu.VMEM((CH,), jnp.int32),
            exA=pltpu.VMEM((CH,), jnp.float32),
            exB=pltpu.VMEM((CH,), jnp.float32),
            rowA=pltpu.VMEM((CH, D), jnp.float32),
            rowB=pltpu.VMEM((CH, D), jnp.float32),
            acc_sh=pltpu.VMEM_SHARED((NPAD, D), jnp.float32),
            den_sh=pltpu.VMEM_SHARED((NPAD,), jnp.float32),
            psem=pltpu.SemaphoreType.DMA,
            gsem=pltpu.SemaphoreType.DMA,
            rsem=pltpu.SemaphoreType.DMA,
            dsem=pltpu.SemaphoreType.DMA,
        ),
    )
    def sc_edge(h, sa, src3, dst3, acc0, acc1, den0, den1,
                asb, adb, srcA, srcB, dstA, dstB, sidxA, sidxB,
                exA, exB, rowA, rowB,
                acc_sh, den_sh, psem, gsem, rsem, dsem):
        cidx = lax.axis_index("c")
        sidx = lax.axis_index("s")
        wid = cidx * 16 + sidx

        pltpu.sync_copy(sa.at[0], asb)
        pltpu.sync_copy(sa.at[1], adb)

        zeros16 = jnp.zeros((16,), jnp.float32)

        def zrow(j, c):
            for k in range(D // 16):
                rowA[j, pl.ds(k * 16, 16)] = zeros16
            return c
        lax.fori_loop(0, CH, zrow, 0)

        def zex(j, c):
            exA[pl.ds(j * 16, 16)] = zeros16
            return c
        lax.fori_loop(0, CH // 16, zex, 0)

        r0 = sidx * ROWS_PER_TILE
        for jb in range(ROWS_PER_TILE // 80):
            rb = pl.multiple_of(r0 + jb * 80, 8)
            pltpu.sync_copy(rowA.at[pl.ds(0, 80)], acc_sh.at[pl.ds(rb, 80)])
            pltpu.sync_copy(exA.at[pl.ds(0, 80)], den_sh.at[pl.ds(rb, 80)])

        m_as = _vmax_all(asb)
        m_ad = _vmax_all(adb)
        msum = m_as + m_ad
        mbound = jnp.where(msum >= 0.0, msum, 0.2 * msum)

        plsc.subcore_barrier()

        esrc = src3.at[wid]
        edst = dst3.at[wid]

        def do_chunk(k, su, du, xu, iu, ru, sv, dv, xv, iv, rv, tail):
            def exgrp(j, c2):
                svv = su[pl.ds(j * 16, 16)]
                dvv = du[pl.ds(j * 16, 16)]
                av = plsc.load_gather(asb, [svv])
                bv = plsc.load_gather(adb, [dvv])
                e = av + bv
                e = jnp.where(e >= 0.0, e, 0.2 * e) - mbound
                xu[pl.ds(j * 16, 16)] = jnp.exp(e)
                return c2
            lax.fori_loop(0, CH // 16, exgrp, 0)

            @pl.when(k == NCHUNK - 1)
            def _():
                for j in range(REAL_GROUPS_LAST, CH // 16):
                    xu[pl.ds(j * 16, 16)] = zeros16

            for j in range(CH // 16):
                iu[pl.ds(j * 16, 16)] = du[pl.ds(j * 16, 16)]

            @pl.when(k >= 1)
            def _():
                pltpu.make_async_copy(rv, acc_sh.at[iv], rsem).wait()
                pltpu.make_async_copy(xv, den_sh.at[iv], dsem).wait()

            if not tail:
                pltpu.make_async_copy(esrc.at[k + 1], sv, psem).wait()
                pltpu.make_async_copy(edst.at[k + 1], dv, psem).wait()
                pltpu.async_copy(h.at[sv], rv, gsem)

            pltpu.make_async_copy(h.at[su], ru, gsem).wait()

            if not tail:
                @pl.when(k + 2 < NCHUNK)
                def _():
                    pltpu.async_copy(esrc.at[k + 2], su, psem)
                    pltpu.async_copy(edst.at[k + 2], du, psem)

            def sblk(q, c2):
                ex16 = xu[pl.ds(q * 16, 16)]
                for jj in range(16):
                    j = q * 16 + jj
                    s_ = ex16[jj]
                    for kk in range(D // 16):
                        ru[j, pl.ds(kk * 16, 16)] = (
                            ru[j, pl.ds(kk * 16, 16)] * s_
                        )
                return c2
            lax.fori_loop(0, CH // 16, sblk, 0)

            pltpu.async_copy(xu, den_sh.at[iu], dsem, add=True)
            pltpu.async_copy(ru, acc_sh.at[iu], rsem, add=True)

        pltpu.async_copy(esrc.at[0], srcA, psem)
        pltpu.async_copy(edst.at[0], dstA, psem)
        pltpu.async_copy(esrc.at[1], srcB, psem)
        pltpu.async_copy(edst.at[1], dstB, psem)
        pltpu.make_async_copy(esrc.at[0], srcA, psem).wait()
        pltpu.make_async_copy(edst.at[0], dstA, psem).wait()
        pltpu.async_copy(h.at[srcA], rowA, gsem)

        def pair(p, c):
            k = p * 2
            do_chunk(k, srcA, dstA, exA, sidxA, rowA,
                     srcB, dstB, exB, sidxB, rowB, False)
            do_chunk(k + 1, srcB, dstB, exB, sidxB, rowB,
                     srcA, dstA, exA, sidxA, rowA, False)
            return c
        lax.fori_loop(0, NCHUNK // 2, pair, 0)

        do_chunk(NCHUNK - 1, srcA, dstA, exA, sidxA, rowA,
                 srcB, dstB, exB, sidxB, rowB, True)
        pltpu.make_async_copy(rowA, acc_sh.at[sidxA], rsem).wait()
        pltpu.make_async_copy(exA, den_sh.at[sidxA], dsem).wait()

        plsc.subcore_barrier()

        rr = pl.multiple_of(r0, 8)

        @pl.when(cidx == 0)
        def _():
            pltpu.sync_copy(acc_sh.at[pl.ds(rr, ROWS_PER_TILE)],
                            acc0.at[pl.ds(rr, ROWS_PER_TILE)])
            pltpu.sync_copy(den_sh.at[pl.ds(rr, ROWS_PER_TILE)],
                            den0.at[pl.ds(rr, ROWS_PER_TILE)])

        @pl.when(cidx == 1)
        def _():
            pltpu.sync_copy(acc_sh.at[pl.ds(rr, ROWS_PER_TILE)],
                            acc1.at[pl.ds(rr, ROWS_PER_TILE)])
            pltpu.sync_copy(den_sh.at[pl.ds(rr, ROWS_PER_TILE)],
                            den1.at[pl.ds(rr, ROWS_PER_TILE)])

    return sc_edge


def kernel(x, edge_index, W0, a_src0, a_dst0, b0, W1, a_src1, a_dst1, b1):
    pad = jnp.zeros((32, PAD_PER_TILE), jnp.int32)
    src3 = jnp.concatenate(
        [edge_index[0].astype(jnp.int32).reshape(32, EDGES_PER_TILE), pad],
        axis=1).reshape(32, NCHUNK, CH)
    dst3 = jnp.concatenate(
        [edge_index[1].astype(jnp.int32).reshape(32, EDGES_PER_TILE), pad],
        axis=1).reshape(32, NCHUNK, CH)
    a20 = jnp.concatenate([a_src0, a_dst0], axis=0)
    a21 = jnp.concatenate([a_src1, a_dst1], axis=0)
    sc_edge = _make_sc_edge()
    h0, sa0 = _head1(x, W0, a20)
    a0, a1, d0, d1 = sc_edge(h0, sa0, src3, dst3)
    h1, sa1 = _mid(a0, a1, d0.reshape(NPAD, 1), d1.reshape(NPAD, 1),
                   b0, W1, a21)
    a0, a1, d0, d1 = sc_edge(h1, sa1, src3, dst3)
    return _final(a0, a1, d0.reshape(NPAD, 1), d1.reshape(NPAD, 1), b1)
